# Initial kernel scaffold; baseline (speedup 1.0000x reference)
import jax
import jax.numpy as jnp
from jax.experimental import pallas as pl


def kernel(psi):
    raise NotImplementedError("write your pallas kernel here")


# speedup vs baseline: 2.4486x; 2.4486x over previous
import jax
import jax.numpy as jnp
from jax.experimental import pallas as pl


_LANES = 128
_BLK = 8192


def _copy_zero_body(nb_in, x_ref, o_ref):
    i = pl.program_id(0)

    @pl.when(i < nb_in)
    def _copy():
        o_ref[...] = x_ref[...]

    @pl.when(i >= nb_in)
    def _zero():
        o_ref[...] = jnp.zeros_like(o_ref)


def kernel(psi):
    rows, cols = psi.shape
    total = rows * cols
    m = total // _LANES
    flat = psi.reshape(m, _LANES)
    nb_in = m // _BLK

    out = pl.pallas_call(
        lambda x_ref, o_ref: _copy_zero_body(nb_in, x_ref, o_ref),
        grid=(2 * nb_in,),
        in_specs=[
            pl.BlockSpec((_BLK, _LANES), lambda i: (jnp.minimum(i, nb_in - 1), 0))
        ],
        out_specs=pl.BlockSpec((_BLK, _LANES), lambda i: (i, 0)),
        out_shape=jax.ShapeDtypeStruct((2 * m, _LANES), psi.dtype),
    )(flat)
    return out.reshape(2 * rows, cols)


# speedup vs baseline: 2.8985x; 1.1837x over previous
import jax
import jax.numpy as jnp
from jax.experimental import pallas as pl


_BLK = 16384


def _copy_zero_body(nb_in, x_ref, o_ref):
    i = pl.program_id(0)

    @pl.when(i < nb_in)
    def _copy():
        o_ref[...] = x_ref[...]

    @pl.when(i >= nb_in)
    def _zero():
        o_ref[...] = jnp.zeros_like(o_ref)


def kernel(psi):
    rows, cols = psi.shape
    nb_in = rows // _BLK

    return pl.pallas_call(
        lambda x_ref, o_ref: _copy_zero_body(nb_in, x_ref, o_ref),
        grid=(2 * nb_in,),
        in_specs=[
            pl.BlockSpec((_BLK, cols), lambda i: (jnp.minimum(i, nb_in - 1), 0))
        ],
        out_specs=pl.BlockSpec((_BLK, cols), lambda i: (i, 0)),
        out_shape=jax.ShapeDtypeStruct((2 * rows, cols), psi.dtype),
    )(psi)
